# Initial kernel scaffold; baseline (speedup 1.0000x reference)
import jax
import jax.numpy as jnp
from jax.experimental import pallas as pl


def kernel(s_logits, t_logits, s_feature, t_feature, y_s, y_t):
    raise NotImplementedError("write your pallas kernel here")


# speedup vs baseline: 3.3894x; 3.3894x over previous
import functools

import jax
import jax.numpy as jnp
from jax import lax
from jax.experimental import pallas as pl
from jax.experimental.pallas import tpu as pltpu
from jax.experimental.pallas import tpu_sc as plsc

_N_CLASS = 1000
_PAD = 1024
_D = 128
_N = 65536
_DECAY = 0.3

_NC = 2
_NS = 16
_NW = _NC * _NS
_ROWS_PER_TILE = _N // _NW
_CHUNK = 128
_NCHUNK = _ROWS_PER_TILE // _CHUNK
_L = 16


_HW = _PAD * _L


def _sc_segment_sums(s_feature, t_feature, ys2d, yt2d, zeros_f):
    mesh = plsc.VectorSubcoreMesh(core_axis_name="c", subcore_axis_name="s")

    @functools.partial(
        pl.kernel,
        out_type=(
            jax.ShapeDtypeStruct((_NC, _PAD, _D), jnp.float32),
            jax.ShapeDtypeStruct((_NC, _PAD, _D), jnp.float32),
            jax.ShapeDtypeStruct((_NW, _HW), jnp.float32),
            jax.ShapeDtypeStruct((_NW, _HW), jnp.float32),
        ),
        mesh=mesh,
        compiler_params=pltpu.CompilerParams(needs_layout_passes=False),
        scratch_types=[
            pltpu.VMEM((_NCHUNK, _CHUNK), jnp.int32),
            pltpu.VMEM((_NCHUNK, _CHUNK), jnp.int32),
            pltpu.VMEM((_CHUNK, _D), jnp.float32),
            pltpu.VMEM((_HW,), jnp.float32),
            pltpu.VMEM((_HW,), jnp.float32),
            pltpu.VMEM_SHARED((_PAD, _D), jnp.float32),
            pltpu.VMEM_SHARED((_PAD, _D), jnp.float32),
        ],
    )
    def k(s_f, t_f, ys, yt, zf,
          s_out, t_out, sc_out, tc_out,
          idx_s, idx_t, fbuf, s_hist, t_hist,
          s_facc, t_facc):
        cid = lax.axis_index("c")
        sid = lax.axis_index("s")
        wid = cid * _NS + sid

        rows = _PAD // _NS
        r0 = sid * rows
        pltpu.sync_copy(zf.at[pl.ds(r0, rows)], s_facc.at[pl.ds(r0, rows)])
        pltpu.sync_copy(zf.at[pl.ds(r0, rows)], t_facc.at[pl.ds(r0, rows)])

        pltpu.sync_copy(ys.at[pl.ds(wid * _NCHUNK, _NCHUNK)], idx_s)
        pltpu.sync_copy(yt.at[pl.ds(wid * _NCHUNK, _NCHUNK)], idx_t)

        def zbody(i, carry):
            s_hist[pl.ds(i * _L, _L)] = jnp.zeros((_L,), jnp.float32)
            t_hist[pl.ds(i * _L, _L)] = jnp.zeros((_L,), jnp.float32)
            return carry

        lax.fori_loop(0, _HW // _L, zbody, 0)
        plsc.subcore_barrier()

        base = wid * _ROWS_PER_TILE
        lane = lax.iota(jnp.int32, _L)
        ones_v = jnp.full((_L,), 1.0, jnp.float32)

        def hist_update(hist, idx_ref, j):
            for kk in range(_CHUNK // _L):
                lbl = idx_ref[j, pl.ds(kk * _L, _L)]
                flat = lax.shift_left(lbl, 4) + lane
                plsc.addupdate_scatter(hist, [flat], ones_v)

        def body(j, carry):
            pltpu.sync_copy(s_f.at[pl.ds(base + j * _CHUNK, _CHUNK)], fbuf)
            pltpu.sync_copy(fbuf, s_facc.at[idx_s.at[j]], add=True)
            hist_update(s_hist, idx_s, j)
            pltpu.sync_copy(t_f.at[pl.ds(base + j * _CHUNK, _CHUNK)], fbuf)
            pltpu.sync_copy(fbuf, t_facc.at[idx_t.at[j]], add=True)
            hist_update(t_hist, idx_t, j)
            return carry

        lax.fori_loop(0, _NCHUNK, body, 0)

        pltpu.sync_copy(s_hist, sc_out.at[wid])
        pltpu.sync_copy(t_hist, tc_out.at[wid])
        plsc.subcore_barrier()
        pltpu.sync_copy(s_facc.at[pl.ds(r0, rows)], s_out.at[cid, pl.ds(r0, rows)])
        pltpu.sync_copy(t_facc.at[pl.ds(r0, rows)], t_out.at[cid, pl.ds(r0, rows)])

    return k(s_feature, t_feature, ys2d, yt2d, zeros_f)


def _unpack_counts(hist_ref):
    h = hist_ref[0]
    for w in range(1, _NW):
        h = h + hist_ref[w]
    return jnp.sum(h, axis=-1, keepdims=True)


def _tc_finalize(s_part, t_part, s_cnt, t_cnt):
    def body(sp_ref, tp_ref, sc_ref, tc_ref, o_ref):
        ssum = sp_ref[0] + sp_ref[1]
        tsum = tp_ref[0] + tp_ref[1]
        scnt = jnp.maximum(_unpack_counts(sc_ref), 1.0)
        tcnt = jnp.maximum(_unpack_counts(tc_ref), 1.0)
        diff = ssum / scnt - tsum / tcnt
        scale = (_DECAY * _DECAY) / (_N_CLASS * _D)
        o_ref[0, 0] = jnp.sum(diff * diff) * scale

    out = pl.pallas_call(
        body,
        out_shape=jax.ShapeDtypeStruct((1, 1), jnp.float32),
        out_specs=pl.BlockSpec(memory_space=pltpu.SMEM),
    )(s_part, t_part, s_cnt, t_cnt)
    return out[0, 0]


def kernel(s_logits, t_logits, s_feature, t_feature, y_s, y_t):
    del s_logits, t_logits
    ys2d = y_s.astype(jnp.int32).reshape(_N // _CHUNK, _CHUNK)
    yt2d = y_t.astype(jnp.int32).reshape(_N // _CHUNK, _CHUNK)
    zeros_f = jnp.zeros((_PAD, _D), jnp.float32)
    s_part, t_part, s_cnt, t_cnt = _sc_segment_sums(
        s_feature, t_feature, ys2d, yt2d, zeros_f)
    s_cnt = s_cnt.reshape(_NW, _PAD, _L)
    t_cnt = t_cnt.reshape(_NW, _PAD, _L)
    return _tc_finalize(s_part, t_part, s_cnt, t_cnt)


# speedup vs baseline: 3.9227x; 1.1574x over previous
import functools

import jax
import jax.numpy as jnp
from jax import lax
from jax.experimental import pallas as pl
from jax.experimental.pallas import tpu as pltpu
from jax.experimental.pallas import tpu_sc as plsc

_N_CLASS = 1000
_PAD = 1024
_D = 128
_N = 65536
_DECAY = 0.3

_NC = 2
_NS = 16
_NW = _NC * _NS
_ROWS_PER_TILE = _N // _NW
_CHUNK = 128
_NCHUNK = _ROWS_PER_TILE // _CHUNK
_L = 16


_HW = _PAD * _L


def _sc_segment_sums(s_feature, t_feature, ys2d, yt2d, zeros_f, zeros_h):
    mesh = plsc.VectorSubcoreMesh(core_axis_name="c", subcore_axis_name="s")

    @functools.partial(
        pl.kernel,
        out_type=(
            jax.ShapeDtypeStruct((_NC, _PAD, _D), jnp.float32),
            jax.ShapeDtypeStruct((_NC, _PAD, _D), jnp.float32),
            jax.ShapeDtypeStruct((_NW, _HW), jnp.float32),
            jax.ShapeDtypeStruct((_NW, _HW), jnp.float32),
        ),
        mesh=mesh,
        compiler_params=pltpu.CompilerParams(needs_layout_passes=False),
        scratch_types=[
            pltpu.VMEM((_NCHUNK, _CHUNK), jnp.int32),
            pltpu.VMEM((_NCHUNK, _CHUNK), jnp.int32),
            pltpu.VMEM((_CHUNK, _D), jnp.float32),
            pltpu.VMEM((_CHUNK, _D), jnp.float32),
            pltpu.VMEM((_HW,), jnp.float32),
            pltpu.VMEM((_HW,), jnp.float32),
            pltpu.VMEM_SHARED((_PAD, _D), jnp.float32),
            pltpu.VMEM_SHARED((_PAD, _D), jnp.float32),
            pltpu.SemaphoreType.DMA,
            pltpu.SemaphoreType.DMA,
            pltpu.SemaphoreType.DMA,
        ],
    )
    def k(s_f, t_f, ys, yt, zf, zh,
          s_out, t_out, sc_out, tc_out,
          idx_s, idx_t, sbuf, tbuf, s_hist, t_hist,
          s_facc, t_facc, gs, gt, ss):
        cid = lax.axis_index("c")
        sid = lax.axis_index("s")
        wid = cid * _NS + sid
        base = wid * _ROWS_PER_TILE

        pltpu.async_copy(s_f.at[pl.ds(base, _CHUNK)], sbuf, gs)

        rows = _PAD // _NS
        r0 = sid * rows
        pltpu.sync_copy(zf.at[pl.ds(r0, rows)], s_facc.at[pl.ds(r0, rows)])
        pltpu.sync_copy(zf.at[pl.ds(r0, rows)], t_facc.at[pl.ds(r0, rows)])

        pltpu.sync_copy(ys.at[pl.ds(wid * _NCHUNK, _NCHUNK)], idx_s)
        pltpu.sync_copy(yt.at[pl.ds(wid * _NCHUNK, _NCHUNK)], idx_t)
        pltpu.sync_copy(zh, s_hist)
        pltpu.sync_copy(zh, t_hist)
        plsc.subcore_barrier()

        lane = lax.iota(jnp.int32, _L)
        ones_v = jnp.full((_L,), 1.0, jnp.float32)

        def hist_update(hist, idx_ref, j):
            for kk in range(_CHUNK // _L):
                lbl = idx_ref[j, pl.ds(kk * _L, _L)]
                flat = lax.shift_left(lbl, 4) + lane
                plsc.addupdate_scatter(hist, [flat], ones_v)

        def body(j, carry):
            pltpu.make_async_copy(s_f.at[pl.ds(base, _CHUNK)], sbuf, gs).wait()
            pltpu.async_copy(t_f.at[pl.ds(base + j * _CHUNK, _CHUNK)], tbuf, gt)
            sc_s = pltpu.async_copy(sbuf, s_facc.at[idx_s.at[j]], ss, add=True)
            hist_update(s_hist, idx_s, j)
            sc_s.wait()
            pltpu.make_async_copy(t_f.at[pl.ds(base, _CHUNK)], tbuf, gt).wait()

            @pl.when(j < _NCHUNK - 1)
            def _():
                pltpu.async_copy(
                    s_f.at[pl.ds(base + (j + 1) * _CHUNK, _CHUNK)], sbuf, gs)

            sc_t = pltpu.async_copy(tbuf, t_facc.at[idx_t.at[j]], ss, add=True)
            hist_update(t_hist, idx_t, j)
            sc_t.wait()
            return carry

        lax.fori_loop(0, _NCHUNK, body, 0)

        pltpu.sync_copy(s_hist, sc_out.at[wid])
        pltpu.sync_copy(t_hist, tc_out.at[wid])
        plsc.subcore_barrier()
        pltpu.sync_copy(s_facc.at[pl.ds(r0, rows)], s_out.at[cid, pl.ds(r0, rows)])
        pltpu.sync_copy(t_facc.at[pl.ds(r0, rows)], t_out.at[cid, pl.ds(r0, rows)])

    return k(s_feature, t_feature, ys2d, yt2d, zeros_f, zeros_h)


def _unpack_counts(hist_ref):
    h = hist_ref[0]
    for w in range(1, _NW):
        h = h + hist_ref[w]
    return jnp.sum(h, axis=-1, keepdims=True)


def _tc_finalize(s_part, t_part, s_cnt, t_cnt):
    def body(sp_ref, tp_ref, sc_ref, tc_ref, o_ref):
        ssum = sp_ref[0] + sp_ref[1]
        tsum = tp_ref[0] + tp_ref[1]
        scnt = jnp.maximum(_unpack_counts(sc_ref), 1.0)
        tcnt = jnp.maximum(_unpack_counts(tc_ref), 1.0)
        diff = ssum / scnt - tsum / tcnt
        scale = (_DECAY * _DECAY) / (_N_CLASS * _D)
        o_ref[0, 0] = jnp.sum(diff * diff) * scale

    out = pl.pallas_call(
        body,
        out_shape=jax.ShapeDtypeStruct((1, 1), jnp.float32),
        out_specs=pl.BlockSpec(memory_space=pltpu.SMEM),
    )(s_part, t_part, s_cnt, t_cnt)
    return out[0, 0]


def kernel(s_logits, t_logits, s_feature, t_feature, y_s, y_t):
    del s_logits, t_logits
    ys2d = y_s.astype(jnp.int32).reshape(_N // _CHUNK, _CHUNK)
    yt2d = y_t.astype(jnp.int32).reshape(_N // _CHUNK, _CHUNK)
    zeros_f = jnp.zeros((_PAD, _D), jnp.float32)
    zeros_h = jnp.zeros((_HW,), jnp.float32)
    s_part, t_part, s_cnt, t_cnt = _sc_segment_sums(
        s_feature, t_feature, ys2d, yt2d, zeros_f, zeros_h)
    s_cnt = s_cnt.reshape(_NW, _PAD, _L)
    t_cnt = t_cnt.reshape(_NW, _PAD, _L)
    return _tc_finalize(s_part, t_part, s_cnt, t_cnt)


# speedup vs baseline: 6.4381x; 1.6412x over previous
import functools

import jax
import jax.numpy as jnp
from jax import lax
from jax.experimental import pallas as pl
from jax.experimental.pallas import tpu as pltpu
from jax.experimental.pallas import tpu_sc as plsc

_N_CLASS = 1000
_PAD = 1024
_D = 128
_N = 65536
_DECAY = 0.3

_NC = 2
_NS = 16
_NW = _NC * _NS
_ROWS_PER_TILE = _N // _NW
_CHUNK = 128
_NCHUNK = _ROWS_PER_TILE // _CHUNK
_L = 16


_HW = _PAD * _L


def _sc_segment_sums(s_feature, t_feature, ys2d, yt2d, zeros_f, iota128):
    mesh = plsc.VectorSubcoreMesh(core_axis_name="c", subcore_axis_name="s")

    @functools.partial(
        pl.kernel,
        out_type=(
            jax.ShapeDtypeStruct((_NC, _PAD, _D), jnp.float32),
            jax.ShapeDtypeStruct((_NC, _PAD, _D), jnp.float32),
            jax.ShapeDtypeStruct((_NC, _CHUNK, _CHUNK), jnp.float32),
            jax.ShapeDtypeStruct((_NC, _CHUNK, _CHUNK), jnp.float32),
        ),
        mesh=mesh,
        compiler_params=pltpu.CompilerParams(needs_layout_passes=False),
        scratch_types=[
            pltpu.VMEM((_NCHUNK, _CHUNK), jnp.int32),
            pltpu.VMEM((_NCHUNK, _CHUNK), jnp.int32),
            pltpu.VMEM((_CHUNK, _D), jnp.float32),
            pltpu.VMEM((_CHUNK, _D), jnp.float32),
            pltpu.VMEM((_CHUNK, _CHUNK), jnp.float32),
            pltpu.VMEM((_CHUNK, _CHUNK), jnp.float32),
            pltpu.VMEM((1, _CHUNK), jnp.int32),
            pltpu.VMEM_SHARED((_PAD, _D), jnp.float32),
            pltpu.VMEM_SHARED((_PAD, _D), jnp.float32),
            pltpu.VMEM_SHARED((_CHUNK, _CHUNK), jnp.float32),
            pltpu.VMEM_SHARED((_CHUNK, _CHUNK), jnp.float32),
            pltpu.SemaphoreType.DMA,
            pltpu.SemaphoreType.DMA,
            pltpu.SemaphoreType.DMA,
        ],
    )
    def k(s_f, t_f, ys, yt, zf, iq,
          s_out, t_out, sc_out, tc_out,
          idx_s, idx_t, sbuf, tbuf, s_hist, t_hist, idq,
          s_facc, t_facc, s_cacc, t_cacc, gs, gt, ss):
        cid = lax.axis_index("c")
        sid = lax.axis_index("s")
        wid = cid * _NS + sid
        base = wid * _ROWS_PER_TILE

        pltpu.async_copy(s_f.at[pl.ds(base, _CHUNK)], sbuf, gs)

        rows = _PAD // _NS
        r0 = sid * rows
        pltpu.sync_copy(zf.at[pl.ds(r0, rows)], s_facc.at[pl.ds(r0, rows)])
        pltpu.sync_copy(zf.at[pl.ds(r0, rows)], t_facc.at[pl.ds(r0, rows)])
        crows = _CHUNK // _NS
        c0 = sid * crows
        pltpu.sync_copy(zf.at[pl.ds(c0, crows)], s_cacc.at[pl.ds(c0, crows)])
        pltpu.sync_copy(zf.at[pl.ds(c0, crows)], t_cacc.at[pl.ds(c0, crows)])

        pltpu.sync_copy(ys.at[pl.ds(wid * _NCHUNK, _NCHUNK)], idx_s)
        pltpu.sync_copy(yt.at[pl.ds(wid * _NCHUNK, _NCHUNK)], idx_t)
        pltpu.sync_copy(iq, idq)
        pltpu.sync_copy(zf.at[pl.ds(0, _CHUNK)], s_hist)
        pltpu.sync_copy(zf.at[pl.ds(0, _CHUNK)], t_hist)
        plsc.subcore_barrier()

        lane = lax.iota(jnp.int32, _L)
        ones_v = jnp.full((_L,), 1.0, jnp.float32)

        def hist_update(hist, idx_ref, j):
            for kk in range(_CHUNK // _L):
                lbl = idx_ref[j, pl.ds(kk * _L, _L)]
                row = lax.shift_right_logical(lbl, 3)
                col = lax.shift_left(jnp.bitwise_and(lbl, 7), 4) + lane
                plsc.addupdate_scatter(hist, [row, col], ones_v)

        def body(j, carry):
            pltpu.make_async_copy(s_f.at[pl.ds(base, _CHUNK)], sbuf, gs).wait()
            pltpu.async_copy(t_f.at[pl.ds(base + j * _CHUNK, _CHUNK)], tbuf, gt)
            sc_s = pltpu.async_copy(sbuf, s_facc.at[idx_s.at[j]], ss, add=True)
            hist_update(s_hist, idx_s, j)
            sc_s.wait()
            pltpu.make_async_copy(t_f.at[pl.ds(base, _CHUNK)], tbuf, gt).wait()

            @pl.when(j < _NCHUNK - 1)
            def _():
                pltpu.async_copy(
                    s_f.at[pl.ds(base + (j + 1) * _CHUNK, _CHUNK)], sbuf, gs)

            sc_t = pltpu.async_copy(tbuf, t_facc.at[idx_t.at[j]], ss, add=True)
            hist_update(t_hist, idx_t, j)
            sc_t.wait()
            return carry

        lax.fori_loop(0, _NCHUNK, body, 0)

        pltpu.sync_copy(s_hist, s_cacc.at[idq.at[0]], add=True)
        pltpu.sync_copy(t_hist, t_cacc.at[idq.at[0]], add=True)
        plsc.subcore_barrier()

        pltpu.sync_copy(s_facc.at[pl.ds(r0, rows)], s_out.at[cid, pl.ds(r0, rows)])
        pltpu.sync_copy(t_facc.at[pl.ds(r0, rows)], t_out.at[cid, pl.ds(r0, rows)])
        pltpu.sync_copy(s_cacc.at[pl.ds(c0, crows)], sc_out.at[cid, pl.ds(c0, crows)])
        pltpu.sync_copy(t_cacc.at[pl.ds(c0, crows)], tc_out.at[cid, pl.ds(c0, crows)])

    return k(s_feature, t_feature, ys2d, yt2d, zeros_f, iota128)


def _unpack_counts(hist_ref):
    h = hist_ref[0] + hist_ref[1]
    cnt = jnp.sum(h.reshape(_CHUNK, 8, _L), axis=-1)
    return cnt.reshape(_PAD, 1)


def _tc_finalize(s_part, t_part, s_cnt, t_cnt):
    def body(sp_ref, tp_ref, sc_ref, tc_ref, o_ref):
        ssum = sp_ref[0] + sp_ref[1]
        tsum = tp_ref[0] + tp_ref[1]
        scnt = jnp.maximum(_unpack_counts(sc_ref), 1.0)
        tcnt = jnp.maximum(_unpack_counts(tc_ref), 1.0)
        diff = ssum / scnt - tsum / tcnt
        scale = (_DECAY * _DECAY) / (_N_CLASS * _D)
        o_ref[0, 0] = jnp.sum(diff * diff) * scale

    out = pl.pallas_call(
        body,
        out_shape=jax.ShapeDtypeStruct((1, 1), jnp.float32),
        out_specs=pl.BlockSpec(memory_space=pltpu.SMEM),
    )(s_part, t_part, s_cnt, t_cnt)
    return out[0, 0]


def kernel(s_logits, t_logits, s_feature, t_feature, y_s, y_t):
    del s_logits, t_logits
    ys2d = y_s.astype(jnp.int32).reshape(_N // _CHUNK, _CHUNK)
    yt2d = y_t.astype(jnp.int32).reshape(_N // _CHUNK, _CHUNK)
    zeros_f = jnp.zeros((_PAD, _D), jnp.float32)
    iota128 = jnp.arange(_CHUNK, dtype=jnp.int32).reshape(1, _CHUNK)
    s_part, t_part, s_cnt, t_cnt = _sc_segment_sums(
        s_feature, t_feature, ys2d, yt2d, zeros_f, iota128)
    return _tc_finalize(s_part, t_part, s_cnt, t_cnt)


# speedup vs baseline: 7.1460x; 1.1100x over previous
import functools

import jax
import jax.numpy as jnp
from jax import lax
from jax.experimental import pallas as pl
from jax.experimental.pallas import tpu as pltpu
from jax.experimental.pallas import tpu_sc as plsc

_N_CLASS = 1000
_PAD = 1024
_D = 128
_N = 65536
_DECAY = 0.3

_NC = 2
_NS = 16
_NW = _NC * _NS
_ROWS_PER_TILE = _N // _NW
_CHUNK = 128
_NCHUNK = _ROWS_PER_TILE // _CHUNK
_L = 16


_HW = _PAD * _L


def _sc_segment_sums(s_feature, t_feature, ys2d, yt2d, zeros_f, iota128):
    mesh = plsc.VectorSubcoreMesh(core_axis_name="c", subcore_axis_name="s")

    @functools.partial(
        pl.kernel,
        out_type=(
            jax.ShapeDtypeStruct((_NC, _PAD, _D), jnp.float32),
            jax.ShapeDtypeStruct((_NC, _PAD, _D), jnp.float32),
            jax.ShapeDtypeStruct((_NC, _CHUNK, _CHUNK), jnp.float32),
            jax.ShapeDtypeStruct((_NC, _CHUNK, _CHUNK), jnp.float32),
        ),
        mesh=mesh,
        compiler_params=pltpu.CompilerParams(needs_layout_passes=False),
        scratch_types=[
            pltpu.VMEM((_NCHUNK, _CHUNK), jnp.int32),
            pltpu.VMEM((_NCHUNK, _CHUNK), jnp.int32),
            pltpu.VMEM((_CHUNK, _D), jnp.float32),
            pltpu.VMEM((_CHUNK, _D), jnp.float32),
            pltpu.VMEM((_CHUNK, _D), jnp.float32),
            pltpu.VMEM((_CHUNK, _D), jnp.float32),
            pltpu.VMEM((_CHUNK, _CHUNK), jnp.float32),
            pltpu.VMEM((_CHUNK, _CHUNK), jnp.float32),
            pltpu.VMEM((1, _CHUNK), jnp.int32),
            pltpu.VMEM_SHARED((_PAD, _D), jnp.float32),
            pltpu.VMEM_SHARED((_PAD, _D), jnp.float32),
            pltpu.VMEM_SHARED((_CHUNK, _CHUNK), jnp.float32),
            pltpu.VMEM_SHARED((_CHUNK, _CHUNK), jnp.float32),
            pltpu.SemaphoreType.DMA,
            pltpu.SemaphoreType.DMA,
            pltpu.SemaphoreType.DMA,
            pltpu.SemaphoreType.DMA,
            pltpu.SemaphoreType.DMA,
            pltpu.SemaphoreType.DMA,
            pltpu.SemaphoreType.DMA,
            pltpu.SemaphoreType.DMA,
        ],
    )
    def k(s_f, t_f, ys, yt, zf, iq,
          s_out, t_out, sc_out, tc_out,
          idx_s, idx_t, sb0, sb1, tb0, tb1, s_hist, t_hist, idq,
          s_facc, t_facc, s_cacc, t_cacc,
          gs0, gs1, gt0, gt1, ss0, ss1, st0, st1):
        cid = lax.axis_index("c")
        sid = lax.axis_index("s")
        wid = cid * _NS + sid
        base = wid * _ROWS_PER_TILE

        pltpu.async_copy(s_f.at[pl.ds(base, _CHUNK)], sb0, gs0)
        pltpu.async_copy(t_f.at[pl.ds(base, _CHUNK)], tb0, gt0)
        pltpu.async_copy(s_f.at[pl.ds(base + _CHUNK, _CHUNK)], sb1, gs1)
        pltpu.async_copy(t_f.at[pl.ds(base + _CHUNK, _CHUNK)], tb1, gt1)

        rows = _PAD // _NS
        r0 = sid * rows
        pltpu.sync_copy(zf.at[pl.ds(r0, rows)], s_facc.at[pl.ds(r0, rows)])
        pltpu.sync_copy(zf.at[pl.ds(r0, rows)], t_facc.at[pl.ds(r0, rows)])
        crows = _CHUNK // _NS
        c0 = sid * crows
        pltpu.sync_copy(zf.at[pl.ds(c0, crows)], s_cacc.at[pl.ds(c0, crows)])
        pltpu.sync_copy(zf.at[pl.ds(c0, crows)], t_cacc.at[pl.ds(c0, crows)])

        pltpu.sync_copy(ys.at[pl.ds(wid * _NCHUNK, _NCHUNK)], idx_s)
        pltpu.sync_copy(yt.at[pl.ds(wid * _NCHUNK, _NCHUNK)], idx_t)
        pltpu.sync_copy(iq, idq)
        pltpu.sync_copy(zf.at[pl.ds(0, _CHUNK)], s_hist)
        pltpu.sync_copy(zf.at[pl.ds(0, _CHUNK)], t_hist)
        plsc.subcore_barrier()

        lane = lax.iota(jnp.int32, _L)
        ones_v = jnp.full((_L,), 1.0, jnp.float32)

        def hist_update(hist, idx_ref, j):
            for kk in range(_CHUNK // _L):
                lbl = idx_ref[j, pl.ds(kk * _L, _L)]
                row = lax.shift_right_logical(lbl, 3)
                col = lax.shift_left(jnp.bitwise_and(lbl, 7), 4) + lane
                plsc.addupdate_scatter(hist, [row, col], ones_v)

        bufs = ((sb0, tb0, gs0, gt0, ss0, st0), (sb1, tb1, gs1, gt1, ss1, st1))

        def body(i, carry):
            for p in range(2):
                j = 2 * i + p
                sb, tb, gs, gt, ss, st = bufs[p]
                pltpu.make_async_copy(s_f.at[pl.ds(base, _CHUNK)], sb, gs).wait()
                pltpu.async_copy(sb, s_facc.at[idx_s.at[j]], ss, add=True)
                pltpu.make_async_copy(t_f.at[pl.ds(base, _CHUNK)], tb, gt).wait()
                pltpu.async_copy(tb, t_facc.at[idx_t.at[j]], st, add=True)
                hist_update(s_hist, idx_s, j)
                hist_update(t_hist, idx_t, j)

            @pl.when(i < _NCHUNK // 2 - 1)
            def _():
                for p in range(2):
                    j = 2 * i + 2 + p
                    sb, tb, gs, gt, ss, st = bufs[p]
                    pltpu.make_async_copy(s_f.at[pl.ds(base, _CHUNK)], sb, ss).wait()
                    pltpu.async_copy(s_f.at[pl.ds(base + j * _CHUNK, _CHUNK)], sb, gs)
                    pltpu.make_async_copy(t_f.at[pl.ds(base, _CHUNK)], tb, st).wait()
                    pltpu.async_copy(t_f.at[pl.ds(base + j * _CHUNK, _CHUNK)], tb, gt)

            return carry

        lax.fori_loop(0, _NCHUNK // 2, body, 0)

        pltpu.make_async_copy(s_f.at[pl.ds(base, _CHUNK)], sb0, ss0).wait()
        pltpu.make_async_copy(s_f.at[pl.ds(base, _CHUNK)], sb1, ss1).wait()
        pltpu.make_async_copy(t_f.at[pl.ds(base, _CHUNK)], tb0, st0).wait()
        pltpu.make_async_copy(t_f.at[pl.ds(base, _CHUNK)], tb1, st1).wait()

        pltpu.sync_copy(s_hist, s_cacc.at[idq.at[0]], add=True)
        pltpu.sync_copy(t_hist, t_cacc.at[idq.at[0]], add=True)
        plsc.subcore_barrier()

        pltpu.sync_copy(s_facc.at[pl.ds(r0, rows)], s_out.at[cid, pl.ds(r0, rows)])
        pltpu.sync_copy(t_facc.at[pl.ds(r0, rows)], t_out.at[cid, pl.ds(r0, rows)])
        pltpu.sync_copy(s_cacc.at[pl.ds(c0, crows)], sc_out.at[cid, pl.ds(c0, crows)])
        pltpu.sync_copy(t_cacc.at[pl.ds(c0, crows)], tc_out.at[cid, pl.ds(c0, crows)])

    return k(s_feature, t_feature, ys2d, yt2d, zeros_f, iota128)


def _unpack_counts(hist_ref):
    h = hist_ref[0] + hist_ref[1]
    cnt = jnp.sum(h.reshape(_CHUNK, 8, _L), axis=-1)
    return cnt.reshape(_PAD, 1)


def _tc_finalize(s_part, t_part, s_cnt, t_cnt):
    def body(sp_ref, tp_ref, sc_ref, tc_ref, o_ref):
        ssum = sp_ref[0] + sp_ref[1]
        tsum = tp_ref[0] + tp_ref[1]
        scnt = jnp.maximum(_unpack_counts(sc_ref), 1.0)
        tcnt = jnp.maximum(_unpack_counts(tc_ref), 1.0)
        diff = ssum / scnt - tsum / tcnt
        scale = (_DECAY * _DECAY) / (_N_CLASS * _D)
        o_ref[0, 0] = jnp.sum(diff * diff) * scale

    out = pl.pallas_call(
        body,
        out_shape=jax.ShapeDtypeStruct((1, 1), jnp.float32),
        out_specs=pl.BlockSpec(memory_space=pltpu.SMEM),
    )(s_part, t_part, s_cnt, t_cnt)
    return out[0, 0]


def kernel(s_logits, t_logits, s_feature, t_feature, y_s, y_t):
    del s_logits, t_logits
    ys2d = y_s.astype(jnp.int32).reshape(_N // _CHUNK, _CHUNK)
    yt2d = y_t.astype(jnp.int32).reshape(_N // _CHUNK, _CHUNK)
    zeros_f = jnp.zeros((_PAD, _D), jnp.float32)
    iota128 = jnp.arange(_CHUNK, dtype=jnp.int32).reshape(1, _CHUNK)
    s_part, t_part, s_cnt, t_cnt = _sc_segment_sums(
        s_feature, t_feature, ys2d, yt2d, zeros_f, iota128)
    return _tc_finalize(s_part, t_part, s_cnt, t_cnt)
